# Initial kernel scaffold; baseline (speedup 1.0000x reference)
import functools

import jax
import jax.numpy as jnp
from jax import lax
from jax.experimental import pallas as pl
from jax.experimental.pallas import tpu as pltpu
from jax.experimental.pallas import tpu_sc as plsc

D_FIELD = 32
CHUNK = 2048


def _gather_kernel(b_per_w, num_cores, chunk):
    n_steps = b_per_w // chunk
    mesh = plsc.VectorSubcoreMesh(core_axis_name="c", subcore_axis_name="s")

    def body(lookup_hbm, table_hbm, out_hbm, idx_v, rows_v, sem):
        wid = lax.axis_index("s") * num_cores + lax.axis_index("c")
        base = wid * b_per_w

        def step(i, carry):
            off = base + i * chunk
            pltpu.sync_copy(lookup_hbm.at[pl.ds(off, chunk)], idx_v)
            pltpu.async_copy(table_hbm.at[idx_v], rows_v, sem).wait()
            pltpu.sync_copy(rows_v, out_hbm.at[pl.ds(off, chunk)])
            return carry

        lax.fori_loop(0, n_steps, step, 0)

    return body, mesh


def kernel(lookup, table):
    batch, hist = lookup.shape
    b_total = batch * hist
    info = plsc.get_sparse_core_info()
    nw = info.num_cores * info.num_subcores
    b_per_w = b_total // nw
    body, mesh = _gather_kernel(b_per_w, info.num_cores, CHUNK)

    flat_idx = lookup.reshape(b_total).astype(jnp.int32)
    out = pl.kernel(
        body,
        out_type=jax.ShapeDtypeStruct((b_total, D_FIELD), jnp.float32),
        mesh=mesh,
        scratch_types=[
            pltpu.VMEM((CHUNK,), jnp.int32),
            pltpu.VMEM((CHUNK, D_FIELD), jnp.float32),
            pltpu.SemaphoreType.DMA,
        ],
    )(flat_idx, table)
    return out.reshape(batch, hist, D_FIELD)


# speedup vs baseline: 4.9492x; 4.9492x over previous
import functools

import jax
import jax.numpy as jnp
from jax import lax
from jax.experimental import pallas as pl
from jax.experimental.pallas import tpu as pltpu
from jax.experimental.pallas import tpu_sc as plsc

D_FIELD = 32
CHUNK = 2048


def _gather_kernel(b_per_w, num_cores, chunk):
    n_steps = b_per_w // chunk
    mesh = plsc.VectorSubcoreMesh(core_axis_name="c", subcore_axis_name="s")

    def body(lookup_hbm, table_hbm, out_hbm, idx_v, rows_v, sem):
        wid = lax.axis_index("s") * num_cores + lax.axis_index("c")
        base = wid * b_per_w

        def step(i, carry):
            off = base + i * chunk
            pltpu.sync_copy(lookup_hbm.at[pl.ds(off, chunk)], idx_v)
            pltpu.async_copy(table_hbm.at[idx_v], rows_v, sem).wait()
            pltpu.sync_copy(rows_v, out_hbm.at[pl.ds(off, chunk)])
            return carry

        lax.fori_loop(0, n_steps, step, 0)

    return body, mesh


def kernel(lookup, table):
    batch, hist = lookup.shape
    b_total = batch * hist
    info = plsc.get_sparse_core_info()
    nw = info.num_cores * info.num_subcores
    b_per_w = b_total // nw
    body, mesh = _gather_kernel(b_per_w, info.num_cores, CHUNK)

    flat_idx = lookup.reshape(b_total).astype(jnp.int32)
    out = pl.kernel(
        body,
        out_type=jax.ShapeDtypeStruct((b_total, D_FIELD), jnp.float32),
        mesh=mesh,
        scratch_types=[
            pltpu.VMEM((CHUNK,), jnp.int32),
            pltpu.VMEM((CHUNK, D_FIELD), jnp.float32),
            pltpu.SemaphoreType.DMA,
        ],
        compiler_params=pltpu.CompilerParams(use_tc_tiling_on_sc=False),
    )(flat_idx, table)
    return out.reshape(batch, hist, D_FIELD)


# speedup vs baseline: 5.0348x; 1.0173x over previous
import jax
import jax.numpy as jnp
from jax import lax
from jax.experimental import pallas as pl
from jax.experimental.pallas import tpu as pltpu
from jax.experimental.pallas import tpu_sc as plsc

D_FIELD = 32
CHUNK = 1024
NBUF = 2


def _gather_kernel(b_per_w, num_cores, chunk, nbuf):
    n_steps = b_per_w // chunk
    n_groups = n_steps // nbuf
    assert n_steps % nbuf == 0 and n_groups >= 3
    mesh = plsc.VectorSubcoreMesh(core_axis_name="c", subcore_axis_name="s")

    def body(lookup_hbm, table_hbm, out_hbm, idx_v, rows_v, *sems):
        sem_i = sems[0:nbuf]
        sem_g = sems[nbuf:2 * nbuf]
        sem_o = sems[2 * nbuf:3 * nbuf]
        wid = lax.axis_index("s") * num_cores + lax.axis_index("c")
        base = wid * b_per_w

        def a_copy(i, b):
            return pltpu.make_async_copy(
                lookup_hbm.at[pl.ds(base + i * chunk, chunk)], idx_v.at[b],
                sem_i[b])

        def b_copy(b):
            return pltpu.make_async_copy(
                table_hbm.at[idx_v.at[b]], rows_v.at[b], sem_g[b])

        def c_copy(i, b):
            return pltpu.make_async_copy(
                rows_v.at[b], out_hbm.at[pl.ds(base + i * chunk, chunk)],
                sem_o[b])

        for b in range(nbuf):
            a_copy(b, b).start()
        for b in range(nbuf):
            a_copy(b, b).wait()
            b_copy(b).start()
            b_copy(b).wait()
            c_copy(b, b).start()
            a_copy(nbuf + b, b).start()

        def group(g, carry):
            for b in range(nbuf):
                i = g * nbuf + b
                c_copy(0, b).wait()
                a_copy(0, b).wait()
                b_copy(b).start()
                b_copy(b).wait()
                c_copy(i, b).start()
                a_copy(i + nbuf, b).start()
            return carry

        lax.fori_loop(1, n_groups - 1, group, 0)

        for b in range(nbuf):
            i = (n_groups - 1) * nbuf + b
            c_copy(0, b).wait()
            a_copy(0, b).wait()
            b_copy(b).start()
            b_copy(b).wait()
            c_copy(i, b).start()
        for b in range(nbuf):
            c_copy(0, b).wait()

    return body, mesh


def kernel(lookup, table):
    batch, hist = lookup.shape
    b_total = batch * hist
    info = plsc.get_sparse_core_info()
    nw = info.num_cores * info.num_subcores
    b_per_w = b_total // nw
    body, mesh = _gather_kernel(b_per_w, info.num_cores, CHUNK, NBUF)

    flat_idx = lookup.reshape(b_total).astype(jnp.int32)
    out = pl.kernel(
        body,
        out_type=jax.ShapeDtypeStruct((b_total, D_FIELD), jnp.float32),
        mesh=mesh,
        scratch_types=[
            pltpu.VMEM((NBUF, CHUNK), jnp.int32),
            pltpu.VMEM((NBUF, CHUNK, D_FIELD), jnp.float32),
        ] + [pltpu.SemaphoreType.DMA] * (3 * NBUF),
        compiler_params=pltpu.CompilerParams(use_tc_tiling_on_sc=False),
    )(flat_idx, table)
    return out.reshape(batch, hist, D_FIELD)


# speedup vs baseline: 5.0393x; 1.0009x over previous
import jax
import jax.numpy as jnp
from jax import lax
from jax.experimental import pallas as pl
from jax.experimental.pallas import tpu as pltpu
from jax.experimental.pallas import tpu_sc as plsc

D_FIELD = 32
ROWS_PER_STEP = 8
NBUF = 2


def _gather_kernel(batch, hist, num_cores, num_subcores):
    nw = num_cores * num_subcores
    rows_per_w = batch // nw
    rp = ROWS_PER_STEP
    nbuf = NBUF
    n_steps = rows_per_w // rp
    n_groups = n_steps // nbuf
    assert rows_per_w % rp == 0 and n_steps % nbuf == 0 and n_groups >= 3
    mesh = plsc.VectorSubcoreMesh(core_axis_name="c", subcore_axis_name="s")

    def body(lookup_hbm, table_hbm, out_hbm, idx_v, rows_v, *sems):
        sem_i = sems[0:nbuf]
        sem_g = sems[nbuf:2 * nbuf]
        sem_o = sems[2 * nbuf:3 * nbuf]
        wid = lax.axis_index("s") * num_cores + lax.axis_index("c")
        base = wid * rows_per_w

        def a_copy(i, b):
            return pltpu.make_async_copy(
                lookup_hbm.at[pl.ds(base + i * rp, rp)], idx_v.at[b],
                sem_i[b])

        def gathers(b):
            return [
                pltpu.make_async_copy(
                    table_hbm.at[idx_v.at[b, r]], rows_v.at[b, r], sem_g[b])
                for r in range(rp)
            ]

        def c_copy(i, b):
            return pltpu.make_async_copy(
                rows_v.at[b], out_hbm.at[pl.ds(base + i * rp, rp)], sem_o[b])

        def fire_drain(b):
            gs = gathers(b)
            for g in gs:
                g.start()
            for g in gs:
                g.wait()

        for b in range(nbuf):
            a_copy(b, b).start()
        for b in range(nbuf):
            a_copy(b, b).wait()
            fire_drain(b)
            c_copy(b, b).start()
            a_copy(nbuf + b, b).start()

        def group(g, carry):
            for b in range(nbuf):
                i = g * nbuf + b
                c_copy(0, b).wait()
                a_copy(0, b).wait()
                fire_drain(b)
                c_copy(i, b).start()
                a_copy(i + nbuf, b).start()
            return carry

        lax.fori_loop(1, n_groups - 1, group, 0)

        for b in range(nbuf):
            i = (n_groups - 1) * nbuf + b
            c_copy(0, b).wait()
            a_copy(0, b).wait()
            fire_drain(b)
            c_copy(i, b).start()
        for b in range(nbuf):
            c_copy(0, b).wait()

    return body, mesh


def kernel(lookup, table):
    batch, hist = lookup.shape
    info = plsc.get_sparse_core_info()
    body, mesh = _gather_kernel(batch, hist, info.num_cores,
                                info.num_subcores)
    out = pl.kernel(
        body,
        out_type=jax.ShapeDtypeStruct((batch, hist, D_FIELD), jnp.float32),
        mesh=mesh,
        scratch_types=[
            pltpu.VMEM((NBUF, ROWS_PER_STEP, hist), jnp.int32),
            pltpu.VMEM((NBUF, ROWS_PER_STEP, hist, D_FIELD), jnp.float32),
        ] + [pltpu.SemaphoreType.DMA] * (3 * NBUF),
        compiler_params=pltpu.CompilerParams(use_tc_tiling_on_sc=False),
    )(lookup.astype(jnp.int32), table)
    return out
